# Initial kernel scaffold; baseline (speedup 1.0000x reference)
import jax
import jax.numpy as jnp
from jax.experimental import pallas as pl


def kernel(edge_feats, segment_ids, W, b, num_graphs):
    raise NotImplementedError("write your pallas kernel here")


# speedup vs baseline: 3.7624x; 3.7624x over previous
import functools

import jax
import jax.numpy as jnp
from jax.experimental import pallas as pl
from jax.experimental.pallas import tpu as pltpu

E = 160000
D = 256
G = 64
BE = 2000
NB = E // BE

_NEG_INF = float("-inf")


def _pass1_body(seg_ref, feats_ref, w_ref, b_ref, e2_ref, h_ref, m_ref, s_ref,
                m_scr, s_scr, acc_scr):
    i = pl.program_id(0)

    @pl.when(i == 0)
    def _init():
        m_scr[...] = jnp.full_like(m_scr, _NEG_INF)
        s_scr[...] = jnp.zeros_like(s_scr)
        acc_scr[...] = jnp.zeros_like(acc_scr)

    feats = feats_ref[...]
    e2 = jnp.dot(feats, w_ref[...], preferred_element_type=jnp.float32)
    e2 = e2 + b_ref[0]
    e2 = jnp.where(e2 >= 0, e2, 0.01 * e2)
    e2_ref[...] = e2

    seg = seg_ref[...]
    gids = jax.lax.broadcasted_iota(jnp.int32, (BE, G), 1)
    mask = seg == gids
    e2m = jnp.where(mask, e2, _NEG_INF)
    bm = jnp.max(e2m, axis=0, keepdims=True)

    m_old = m_scr[...]
    m_new = jnp.maximum(m_old, bm)
    m_safe = jnp.where(m_new == _NEG_INF, 0.0, m_new)
    factor = jnp.where(m_new == _NEG_INF, 0.0, jnp.exp(m_old - m_new))
    p = jnp.exp(e2m - m_safe)

    m_scr[...] = m_new
    s_scr[...] = s_scr[...] * factor + jnp.sum(p, axis=0, keepdims=True)
    pf = jax.lax.dot_general(p, feats, (((0,), (0,)), ((), ())),
                             preferred_element_type=jnp.float32)
    acc_scr[...] = acc_scr[...] * factor.reshape(G, 1) + pf

    @pl.when(i == NB - 1)
    def _finish():
        m_ref[...] = m_scr[...]
        s = s_scr[...]
        s_ref[...] = s
        sc = s.reshape(G, 1)
        h_ref[...] = jnp.where(sc > 0, acc_scr[...] / sc, 0.0)


def _pass2_body(seg_ref, e2_ref, m_ref, s_ref, w_out_ref):
    seg = seg_ref[...]
    e2 = e2_ref[...]
    gids = jax.lax.broadcasted_iota(jnp.int32, (BE, G), 1)
    mask = seg == gids
    m_e = jnp.sum(jnp.where(mask, m_ref[...], 0.0), axis=1, keepdims=True)
    s_e = jnp.sum(jnp.where(mask, s_ref[...], 0.0), axis=1, keepdims=True)
    w_out_ref[...] = jnp.exp(e2 - m_e) / s_e


@functools.partial(jax.jit, static_argnames=("interpret",))
def _run(edge_feats, segment_ids, W, b, interpret=False):
    seg2 = segment_ids.astype(jnp.int32).reshape(E, 1)

    e2, h, m, s = pl.pallas_call(
        _pass1_body,
        grid=(NB,),
        in_specs=[
            pl.BlockSpec((BE, 1), lambda i: (i, 0)),
            pl.BlockSpec((BE, D), lambda i: (i, 0)),
            pl.BlockSpec((D, 1), lambda i: (0, 0)),
            pl.BlockSpec(memory_space=pltpu.SMEM),
        ],
        out_specs=[
            pl.BlockSpec((BE, 1), lambda i: (i, 0)),
            pl.BlockSpec((G, D), lambda i: (0, 0)),
            pl.BlockSpec((1, G), lambda i: (0, 0)),
            pl.BlockSpec((1, G), lambda i: (0, 0)),
        ],
        out_shape=[
            jax.ShapeDtypeStruct((E, 1), jnp.float32),
            jax.ShapeDtypeStruct((G, D), jnp.float32),
            jax.ShapeDtypeStruct((1, G), jnp.float32),
            jax.ShapeDtypeStruct((1, G), jnp.float32),
        ],
        scratch_shapes=[
            pltpu.VMEM((1, G), jnp.float32),
            pltpu.VMEM((1, G), jnp.float32),
            pltpu.VMEM((G, D), jnp.float32),
        ],
        interpret=interpret,
    )(seg2, edge_feats, W, b)

    w = pl.pallas_call(
        _pass2_body,
        grid=(NB,),
        in_specs=[
            pl.BlockSpec((BE, 1), lambda i: (i, 0)),
            pl.BlockSpec((BE, 1), lambda i: (i, 0)),
            pl.BlockSpec((1, G), lambda i: (0, 0)),
            pl.BlockSpec((1, G), lambda i: (0, 0)),
        ],
        out_specs=pl.BlockSpec((BE, 1), lambda i: (i, 0)),
        out_shape=jax.ShapeDtypeStruct((E, 1), jnp.float32),
        interpret=interpret,
    )(seg2, e2, m, s)

    return h, w


def kernel(edge_feats, segment_ids, W, b, num_graphs):
    del num_graphs
    return _run(edge_feats, segment_ids, W, b)


# speedup vs baseline: 6.3860x; 1.6973x over previous
import functools

import jax
import jax.numpy as jnp
from jax.experimental import pallas as pl
from jax.experimental.pallas import tpu as pltpu

E = 160000
D = 256
G = 64
BE = 2000
NB = E // BE

_NEG_INF = float("-inf")


def _pass1_body(seg_ref, feats_ref, w_ref, b_ref, e2_ref, h_ref, c_ref,
                m_scr, s_scr, acc_scr):
    i = pl.program_id(0)

    @pl.when(i == 0)
    def _init():
        m_scr[...] = jnp.full_like(m_scr, _NEG_INF)
        s_scr[...] = jnp.zeros_like(s_scr)
        acc_scr[...] = jnp.zeros_like(acc_scr)

    feats = feats_ref[...]
    e2 = jnp.dot(feats, w_ref[...], preferred_element_type=jnp.float32)
    e2 = e2 + b_ref[0]
    e2 = jnp.where(e2 >= 0, e2, 0.01 * e2)
    e2_ref[0] = jax.lax.transpose(e2, (1, 0))

    seg = jax.lax.transpose(seg_ref[0], (1, 0))
    gids = jax.lax.broadcasted_iota(jnp.int32, (BE, G), 1)
    mask = seg == gids
    e2m = jnp.where(mask, e2, _NEG_INF)
    bm = jnp.max(e2m, axis=0, keepdims=True)

    m_old = m_scr[...]
    m_new = jnp.maximum(m_old, bm)
    m_safe = jnp.where(m_new == _NEG_INF, 0.0, m_new)
    factor = jnp.where(m_new == _NEG_INF, 0.0, jnp.exp(m_old - m_new))
    p = jnp.exp(e2m - m_safe)

    m_scr[...] = m_new
    s_scr[...] = s_scr[...] * factor + jnp.sum(p, axis=0, keepdims=True)
    pf = jax.lax.dot_general(p, feats, (((0,), (0,)), ((), ())),
                             preferred_element_type=jnp.float32)
    acc_scr[...] = acc_scr[...] * factor.reshape(G, 1) + pf

    @pl.when(i == NB - 1)
    def _finish():
        s = s_scr[...]
        m = m_scr[...]
        c_ref[...] = jnp.where(s > 0, m + jnp.log(s), 0.0)
        sc = s.reshape(G, 1)
        h_ref[...] = jnp.where(sc > 0, acc_scr[...] / sc, 0.0)


def _pass2_body(seg_ref, e2_ref, c_ref, w_out_ref):
    seg = seg_ref[0]
    e2 = e2_ref[0]
    c = c_ref[...]
    gids = jax.lax.broadcasted_iota(jnp.int32, (G, BE), 0)
    mask = seg == gids
    c_e = jnp.sum(jnp.where(mask, c, 0.0), axis=0, keepdims=True)
    w_out_ref[0] = jnp.exp(e2 - c_e)


@functools.partial(jax.jit, static_argnames=("interpret",))
def _run(edge_feats, segment_ids, W, b, interpret=False):
    seg3 = segment_ids.astype(jnp.int32).reshape(NB, 1, BE)

    e23, h, c = pl.pallas_call(
        _pass1_body,
        grid=(NB,),
        in_specs=[
            pl.BlockSpec((1, 1, BE), lambda i: (i, 0, 0)),
            pl.BlockSpec((BE, D), lambda i: (i, 0)),
            pl.BlockSpec((D, 1), lambda i: (0, 0)),
            pl.BlockSpec(memory_space=pltpu.SMEM),
        ],
        out_specs=[
            pl.BlockSpec((1, 1, BE), lambda i: (i, 0, 0)),
            pl.BlockSpec((G, D), lambda i: (0, 0)),
            pl.BlockSpec((1, G), lambda i: (0, 0)),
        ],
        out_shape=[
            jax.ShapeDtypeStruct((NB, 1, BE), jnp.float32),
            jax.ShapeDtypeStruct((G, D), jnp.float32),
            jax.ShapeDtypeStruct((1, G), jnp.float32),
        ],
        scratch_shapes=[
            pltpu.VMEM((1, G), jnp.float32),
            pltpu.VMEM((1, G), jnp.float32),
            pltpu.VMEM((G, D), jnp.float32),
        ],
        interpret=interpret,
    )(seg3, edge_feats, W, b)

    w3 = pl.pallas_call(
        _pass2_body,
        grid=(NB,),
        in_specs=[
            pl.BlockSpec((1, 1, BE), lambda i: (i, 0, 0)),
            pl.BlockSpec((1, 1, BE), lambda i: (i, 0, 0)),
            pl.BlockSpec((G, 1), lambda i: (0, 0)),
        ],
        out_specs=pl.BlockSpec((1, 1, BE), lambda i: (i, 0, 0)),
        out_shape=jax.ShapeDtypeStruct((NB, 1, BE), jnp.float32),
        interpret=interpret,
    )(seg3, e23, c.reshape(G, 1))

    return h, w3.reshape(E, 1)


def kernel(edge_feats, segment_ids, W, b, num_graphs):
    del num_graphs
    return _run(edge_feats, segment_ids, W, b)


# speedup vs baseline: 11.9487x; 1.8711x over previous
import functools

import jax
import jax.numpy as jnp
from jax.experimental import pallas as pl
from jax.experimental.pallas import tpu as pltpu

E = 160000
D = 256
G = 64
BE = 4000
NB = E // BE

_NEG_INF = float("-inf")


def _pass1_body(seg_ref, feats_ref, wt_ref, b_ref, e2_ref, h_ref, c_ref,
                m_scr, s_scr, acc_scr):
    i = pl.program_id(0)

    @pl.when(i == 0)
    def _init():
        m_scr[...] = jnp.full_like(m_scr, _NEG_INF)
        s_scr[...] = jnp.zeros_like(s_scr)
        acc_scr[...] = jnp.zeros_like(acc_scr)

    feats = feats_ref[...]
    e2 = jax.lax.dot_general(wt_ref[...], feats, (((1,), (1,)), ((), ())),
                             preferred_element_type=jnp.float32)
    e2 = e2 + b_ref[0]
    e2 = jnp.where(e2 >= 0, e2, 0.01 * e2)
    e2_ref[0] = e2

    seg = seg_ref[0]
    gids = jax.lax.broadcasted_iota(jnp.int32, (G, BE), 0)
    mask = seg == gids
    e2m = jnp.where(mask, e2, _NEG_INF)
    bm = jnp.max(e2m, axis=1, keepdims=True)

    m_old = m_scr[...]
    m_new = jnp.maximum(m_old, bm)
    m_safe = jnp.where(m_new == _NEG_INF, 0.0, m_new)
    factor = jnp.where(m_new == _NEG_INF, 0.0, jnp.exp(m_old - m_new))

    m_e = jnp.sum(jnp.where(mask, m_safe, 0.0), axis=0, keepdims=True)
    p = jnp.exp(e2 - m_e)
    p2 = jnp.where(mask, p, 0.0)

    m_scr[...] = m_new
    s_scr[...] = s_scr[...] * factor + jnp.sum(p2, axis=1, keepdims=True)
    pf = jax.lax.dot_general(p2, feats, (((1,), (0,)), ((), ())),
                             preferred_element_type=jnp.float32)
    acc_scr[...] = acc_scr[...] * factor + pf

    @pl.when(i == NB - 1)
    def _finish():
        s = s_scr[...]
        m = m_scr[...]
        c_ref[...] = jnp.where(s > 0, m + jnp.log(s), 0.0)
        h_ref[...] = jnp.where(s > 0, acc_scr[...] / s, 0.0)


def _pass2_body(seg_ref, e2_ref, c_ref, w_out_ref):
    seg = seg_ref[0]
    e2 = e2_ref[0]
    c = c_ref[...]
    gids = jax.lax.broadcasted_iota(jnp.int32, (G, BE), 0)
    mask = seg == gids
    c_e = jnp.sum(jnp.where(mask, c, 0.0), axis=0, keepdims=True)
    w_out_ref[0] = jnp.exp(e2 - c_e)


@functools.partial(jax.jit, static_argnames=("interpret",))
def _run(edge_feats, segment_ids, W, b, interpret=False):
    seg3 = segment_ids.astype(jnp.int32).reshape(NB, 1, BE)
    wt = W.reshape(1, D)

    e23, h, c = pl.pallas_call(
        _pass1_body,
        grid=(NB,),
        in_specs=[
            pl.BlockSpec((1, 1, BE), lambda i: (i, 0, 0)),
            pl.BlockSpec((BE, D), lambda i: (i, 0)),
            pl.BlockSpec((1, D), lambda i: (0, 0)),
            pl.BlockSpec(memory_space=pltpu.SMEM),
        ],
        out_specs=[
            pl.BlockSpec((1, 1, BE), lambda i: (i, 0, 0)),
            pl.BlockSpec((G, D), lambda i: (0, 0)),
            pl.BlockSpec((G, 1), lambda i: (0, 0)),
        ],
        out_shape=[
            jax.ShapeDtypeStruct((NB, 1, BE), jnp.float32),
            jax.ShapeDtypeStruct((G, D), jnp.float32),
            jax.ShapeDtypeStruct((G, 1), jnp.float32),
        ],
        scratch_shapes=[
            pltpu.VMEM((G, 1), jnp.float32),
            pltpu.VMEM((G, 1), jnp.float32),
            pltpu.VMEM((G, D), jnp.float32),
        ],
        interpret=interpret,
    )(seg3, edge_feats, wt, b)

    w3 = pl.pallas_call(
        _pass2_body,
        grid=(NB,),
        in_specs=[
            pl.BlockSpec((1, 1, BE), lambda i: (i, 0, 0)),
            pl.BlockSpec((1, 1, BE), lambda i: (i, 0, 0)),
            pl.BlockSpec((G, 1), lambda i: (0, 0)),
        ],
        out_specs=pl.BlockSpec((1, 1, BE), lambda i: (i, 0, 0)),
        out_shape=jax.ShapeDtypeStruct((NB, 1, BE), jnp.float32),
        interpret=interpret,
    )(seg3, e23, c)

    return h, w3.reshape(E, 1)


def kernel(edge_feats, segment_ids, W, b, num_graphs):
    del num_graphs
    return _run(edge_feats, segment_ids, W, b)


# speedup vs baseline: 14.8958x; 1.2466x over previous
import functools

import jax
import jax.numpy as jnp
from jax.experimental import pallas as pl
from jax.experimental.pallas import tpu as pltpu

E = 160000
D = 256
G = 64
BE = 8000
NB = E // BE

_NEG_INF = float("-inf")


def _pass1_body(seg_ref, feats_ref, wt_ref, b_ref, e2_ref, h_ref, c_ref,
                m_scr, s_scr, acc_scr):
    i = pl.program_id(0)

    @pl.when(i == 0)
    def _init():
        m_scr[...] = jnp.full_like(m_scr, _NEG_INF)
        s_scr[...] = jnp.zeros_like(s_scr)
        acc_scr[...] = jnp.zeros_like(acc_scr)

    feats = feats_ref[...]
    e2 = jax.lax.dot_general(wt_ref[...], feats, (((1,), (1,)), ((), ())),
                             preferred_element_type=jnp.float32)
    e2 = e2 + b_ref[0]
    e2 = jnp.where(e2 >= 0, e2, 0.01 * e2)
    e2_ref[0] = e2

    seg = seg_ref[0]
    gids = jax.lax.broadcasted_iota(jnp.int32, (G, BE), 0)
    mask = seg == gids
    e2m = jnp.where(mask, e2, _NEG_INF)
    bm = jnp.max(e2m, axis=1, keepdims=True)

    m_old = m_scr[...]
    m_new = jnp.maximum(m_old, bm)
    m_safe = jnp.where(m_new == _NEG_INF, 0.0, m_new)
    factor = jnp.where(m_new == _NEG_INF, 0.0, jnp.exp(m_old - m_new))

    m_e = jnp.sum(jnp.where(mask, m_safe, 0.0), axis=0, keepdims=True)
    p = jnp.exp(e2 - m_e)
    p2 = jnp.where(mask, p, 0.0)

    m_scr[...] = m_new
    s_scr[...] = s_scr[...] * factor + jnp.sum(p2, axis=1, keepdims=True)
    pf = jax.lax.dot_general(p2, feats, (((1,), (0,)), ((), ())),
                             preferred_element_type=jnp.float32)
    acc_scr[...] = acc_scr[...] * factor + pf

    @pl.when(i == NB - 1)
    def _finish():
        s = s_scr[...]
        m = m_scr[...]
        c_ref[...] = jnp.where(s > 0, m + jnp.log(s), 0.0)
        h_ref[...] = jnp.where(s > 0, acc_scr[...] / s, 0.0)


def _pass2_body(seg_ref, e2_ref, c_ref, w_out_ref):
    seg = seg_ref[0]
    e2 = e2_ref[0]
    c = c_ref[...]
    gids = jax.lax.broadcasted_iota(jnp.int32, (G, BE), 0)
    mask = seg == gids
    c_e = jnp.sum(jnp.where(mask, c, 0.0), axis=0, keepdims=True)
    w_out_ref[0] = jnp.exp(e2 - c_e)


@functools.partial(jax.jit, static_argnames=("interpret",))
def _run(edge_feats, segment_ids, W, b, interpret=False):
    seg3 = segment_ids.astype(jnp.int32).reshape(NB, 1, BE)
    wt = W.reshape(1, D)

    e23, h, c = pl.pallas_call(
        _pass1_body,
        grid=(NB,),
        in_specs=[
            pl.BlockSpec((1, 1, BE), lambda i: (i, 0, 0)),
            pl.BlockSpec((BE, D), lambda i: (i, 0)),
            pl.BlockSpec((1, D), lambda i: (0, 0)),
            pl.BlockSpec(memory_space=pltpu.SMEM),
        ],
        out_specs=[
            pl.BlockSpec((1, 1, BE), lambda i: (i, 0, 0)),
            pl.BlockSpec((G, D), lambda i: (0, 0)),
            pl.BlockSpec((G, 1), lambda i: (0, 0)),
        ],
        out_shape=[
            jax.ShapeDtypeStruct((NB, 1, BE), jnp.float32),
            jax.ShapeDtypeStruct((G, D), jnp.float32),
            jax.ShapeDtypeStruct((G, 1), jnp.float32),
        ],
        scratch_shapes=[
            pltpu.VMEM((G, 1), jnp.float32),
            pltpu.VMEM((G, 1), jnp.float32),
            pltpu.VMEM((G, D), jnp.float32),
        ],
        interpret=interpret,
    )(seg3, edge_feats, wt, b)

    w3 = pl.pallas_call(
        _pass2_body,
        grid=(NB,),
        in_specs=[
            pl.BlockSpec((1, 1, BE), lambda i: (i, 0, 0)),
            pl.BlockSpec((1, 1, BE), lambda i: (i, 0, 0)),
            pl.BlockSpec((G, 1), lambda i: (0, 0)),
        ],
        out_specs=pl.BlockSpec((1, 1, BE), lambda i: (i, 0, 0)),
        out_shape=jax.ShapeDtypeStruct((NB, 1, BE), jnp.float32),
        interpret=interpret,
    )(seg3, e23, c)

    return h, w3.reshape(E, 1)


def kernel(edge_feats, segment_ids, W, b, num_graphs):
    del num_graphs
    return _run(edge_feats, segment_ids, W, b)


# speedup vs baseline: 16.7027x; 1.1213x over previous
import functools

import jax
import jax.numpy as jnp
from jax.experimental import pallas as pl
from jax.experimental.pallas import tpu as pltpu

E = 160000
D = 256
G = 64
BE = 16000
NB = E // BE

_NEG_INF = float("-inf")


def _pass1_body(seg_ref, feats_ref, wt_ref, b_ref, e2_ref, h_ref, c_ref,
                m_scr, s_scr, acc_scr):
    i = pl.program_id(0)

    @pl.when(i == 0)
    def _init():
        m_scr[...] = jnp.full_like(m_scr, _NEG_INF)
        s_scr[...] = jnp.zeros_like(s_scr)
        acc_scr[...] = jnp.zeros_like(acc_scr)

    feats = feats_ref[...]
    e2 = jax.lax.dot_general(wt_ref[...], feats, (((1,), (1,)), ((), ())),
                             preferred_element_type=jnp.float32)
    e2 = e2 + b_ref[0]
    e2 = jnp.where(e2 >= 0, e2, 0.01 * e2)
    e2_ref[0] = e2

    seg = seg_ref[0]
    gids = jax.lax.broadcasted_iota(jnp.int32, (G, BE), 0)
    mask = seg == gids
    e2m = jnp.where(mask, e2, _NEG_INF)
    bm = jnp.max(e2m, axis=1, keepdims=True)

    m_old = m_scr[...]
    m_new = jnp.maximum(m_old, bm)
    m_safe = jnp.where(m_new == _NEG_INF, 0.0, m_new)
    factor = jnp.where(m_new == _NEG_INF, 0.0, jnp.exp(m_old - m_new))

    m_e = jnp.sum(jnp.where(mask, m_safe, 0.0), axis=0, keepdims=True)
    p = jnp.exp(e2 - m_e)
    p2 = jnp.where(mask, p, 0.0)

    m_scr[...] = m_new
    s_scr[...] = s_scr[...] * factor + jnp.sum(p2, axis=1, keepdims=True)
    pf = jax.lax.dot_general(p2, feats, (((1,), (0,)), ((), ())),
                             preferred_element_type=jnp.float32)
    acc_scr[...] = acc_scr[...] * factor + pf

    @pl.when(i == NB - 1)
    def _finish():
        s = s_scr[...]
        m = m_scr[...]
        c_ref[...] = jnp.where(s > 0, m + jnp.log(s), 0.0)
        h_ref[...] = jnp.where(s > 0, acc_scr[...] / s, 0.0)


def _pass2_body(seg_ref, e2_ref, c_ref, w_out_ref):
    seg = seg_ref[0]
    e2 = e2_ref[0]
    c = c_ref[...]
    gids = jax.lax.broadcasted_iota(jnp.int32, (G, BE), 0)
    mask = seg == gids
    c_e = jnp.sum(jnp.where(mask, c, 0.0), axis=0, keepdims=True)
    w_out_ref[0] = jnp.exp(e2 - c_e)


@functools.partial(jax.jit, static_argnames=("interpret",))
def _run(edge_feats, segment_ids, W, b, interpret=False):
    seg3 = segment_ids.astype(jnp.int32).reshape(NB, 1, BE)
    wt = W.reshape(1, D)

    e23, h, c = pl.pallas_call(
        _pass1_body,
        grid=(NB,),
        in_specs=[
            pl.BlockSpec((1, 1, BE), lambda i: (i, 0, 0)),
            pl.BlockSpec((BE, D), lambda i: (i, 0)),
            pl.BlockSpec((1, D), lambda i: (0, 0)),
            pl.BlockSpec(memory_space=pltpu.SMEM),
        ],
        out_specs=[
            pl.BlockSpec((1, 1, BE), lambda i: (i, 0, 0)),
            pl.BlockSpec((G, D), lambda i: (0, 0)),
            pl.BlockSpec((G, 1), lambda i: (0, 0)),
        ],
        out_shape=[
            jax.ShapeDtypeStruct((NB, 1, BE), jnp.float32),
            jax.ShapeDtypeStruct((G, D), jnp.float32),
            jax.ShapeDtypeStruct((G, 1), jnp.float32),
        ],
        scratch_shapes=[
            pltpu.VMEM((G, 1), jnp.float32),
            pltpu.VMEM((G, 1), jnp.float32),
            pltpu.VMEM((G, D), jnp.float32),
        ],
        interpret=interpret,
    )(seg3, edge_feats, wt, b)

    w3 = pl.pallas_call(
        _pass2_body,
        grid=(NB,),
        in_specs=[
            pl.BlockSpec((1, 1, BE), lambda i: (i, 0, 0)),
            pl.BlockSpec((1, 1, BE), lambda i: (i, 0, 0)),
            pl.BlockSpec((G, 1), lambda i: (0, 0)),
        ],
        out_specs=pl.BlockSpec((1, 1, BE), lambda i: (i, 0, 0)),
        out_shape=jax.ShapeDtypeStruct((NB, 1, BE), jnp.float32),
        interpret=interpret,
    )(seg3, e23, c)

    return h, w3.reshape(E, 1)


def kernel(edge_feats, segment_ids, W, b, num_graphs):
    del num_graphs
    return _run(edge_feats, segment_ids, W, b)
